# baseline (speedup 1.0000x reference)
import functools
import math

import jax
import jax.numpy as jnp
from jax import lax
from jax.experimental import pallas as pl
from jax.experimental.pallas import tpu as pltpu
from jax.experimental.pallas import tpu_sc as plsc

_SIZE = 100000
_SMOOTHING = 0.1
_CONF = 1.0 - _SMOOTHING
_N = 1024
_FILL = _SMOOTHING / (_SIZE - 1)
_CONST = _N * ((_SIZE - 1) * _FILL * math.log(_FILL) + _CONF * math.log(_CONF))

_NC, _NS = 2, 16
_NW = _NC * _NS

_R_SC = 1024
_R_TC = _N - _R_SC
_C_ALIGN = 99968
_RPW = _R_SC // _NW
_NSTRIPE = _RPW // 8

_CH_SIZES = [3200] * 30 + [3968]
_CH_OFFS = [sum(_CH_SIZES[:k]) for k in range(len(_CH_SIZES))]
_CH_MAX = max(_CH_SIZES)


def _tc_body(sliv_ref, tgt2_ref, o_ref):
    x2 = sliv_ref[...]
    t2 = tgt2_ref[...]
    cols2 = jax.lax.broadcasted_iota(jnp.int32, x2.shape, 1) + _C_ALIGN
    rows2 = jax.lax.broadcasted_iota(jnp.int32, x2.shape, 0)
    dense = jnp.sum(jnp.where(rows2 >= _R_TC, x2, jnp.float32(0.0)))
    corr = jnp.sum(jnp.where(cols2 == t2, x2, jnp.float32(0.0)))
    o_ref[0, 0] = (jnp.float32(_CONST)
                   - jnp.float32(_FILL) * dense
                   - jnp.float32(_CONF - _FILL) * corr)


def _tc_sum(sliver, tgt2d):
    return pl.pallas_call(
        _tc_body,
        out_specs=pl.BlockSpec(memory_space=pltpu.SMEM),
        out_shape=jax.ShapeDtypeStruct((1, 1), jnp.float32),
    )(sliver, tgt2d)


_sc_mesh = plsc.VectorSubcoreMesh(core_axis_name="c", subcore_axis_name="s")


@functools.partial(
    pl.kernel,
    mesh=_sc_mesh,
    out_type=jax.ShapeDtypeStruct((_NW, 16), jnp.float32),
    scratch_types=[
        pltpu.VMEM((48,), jnp.int32),
        pltpu.VMEM((2, 8, _CH_MAX), jnp.float32),
        pltpu.VMEM((16,), jnp.float32),
        pltpu.SemaphoreType.DMA,
        pltpu.SemaphoreType.DMA,
    ],
)
def _sc_part(x_hbm, tgt_hbm, out_hbm, tbuf, buf, stage, sem0, sem1):
    wid = lax.axis_index("s") * _NC + lax.axis_index("c")
    sems = (sem0, sem1)
    r0 = _R_TC + _RPW * wid

    pltpu.sync_copy(tgt_hbm.at[pl.ds(r0, _RPW)], tbuf.at[pl.ds(0, _RPW)])

    rows16 = lax.iota(jnp.int32, 16)
    zero16 = jnp.zeros((16,), jnp.float32)

    def _stripe(s, carry):
        g = carry[8]
        accs = carry[:8]
        rs = pl.multiple_of(r0 + 8 * s, 8)
        tv = tbuf[pl.ds(pl.multiple_of(8 * s, 8), 16)]
        ts = [tv[r] for r in range(8)]

        def _start(k):
            pltpu.async_copy(
                x_hbm.at[pl.ds(rs, 8), pl.ds(_CH_OFFS[k], _CH_SIZES[k])],
                buf.at[k % 2, :, pl.ds(0, _CH_SIZES[k])], sems[k % 2])

        _start(0)
        for k in range(len(_CH_SIZES)):
            if k + 1 < len(_CH_SIZES):
                _start(k + 1)
            slot = k % 2
            off, ch = _CH_OFFS[k], _CH_SIZES[k]
            pltpu.make_async_copy(
                x_hbm.at[pl.ds(rs, 8), pl.ds(off, ch)],
                buf.at[slot, :, pl.ds(0, ch)], sems[slot]).wait()

            def _ibody(jj, accs, slot=slot):
                jx = pl.multiple_of(jj * 32, 32)
                out = []
                for r in range(8):
                    out.append(accs[r] + buf[slot, r, pl.ds(jx, 16)]
                               + buf[slot, r, pl.ds(jx + 16, 16)])
                return tuple(out)

            accs = lax.fori_loop(0, ch // 32, _ibody, accs)

            for r in range(8):
                dt = ts[r] - jnp.int32(off)
                l0 = jnp.minimum(
                    jnp.maximum(dt & jnp.int32(-16), jnp.int32(0)),
                    jnp.int32(ch - 16))
                v = buf[slot, r, pl.ds(pl.multiple_of(l0, 16), 16)]
                lsel = jnp.where(dt >= 0,
                                 jnp.where(dt < ch, dt & jnp.int32(15),
                                           jnp.int32(16)),
                                 jnp.int32(16))
                g = g + jnp.where(rows16 == jnp.full((16,), lsel), v, zero16)
        return accs + (g,)

    init = tuple(jnp.zeros((16,), jnp.float32) for _ in range(9))
    res = lax.fori_loop(0, _NSTRIPE, _stripe, init)
    acc = ((res[0] + res[1]) + (res[2] + res[3])
           + ((res[4] + res[5]) + (res[6] + res[7])))
    stage[...] = (jnp.float32(_FILL) * acc
                  + jnp.float32(_CONF - _FILL) * res[8])
    pltpu.sync_copy(stage, out_hbm.at[wid])


def kernel(x, target):
    tgt = target.astype(jnp.int32)
    sliver = lax.slice(x, (0, _C_ALIGN), (_N, _SIZE))
    tc_out = _tc_sum(sliver, tgt.reshape(_N, 1))
    sc_out = _sc_part(x, tgt)
    return (tc_out[0, 0] - jnp.sum(sc_out)).reshape(())


# speedup vs baseline: 1.0315x; 1.0315x over previous
import functools
import math

import jax
import jax.numpy as jnp
from jax import lax
from jax.experimental import pallas as pl
from jax.experimental.pallas import tpu as pltpu
from jax.experimental.pallas import tpu_sc as plsc

_SIZE = 100000
_SMOOTHING = 0.1
_CONF = 1.0 - _SMOOTHING
_N = 1024
_FILL = _SMOOTHING / (_SIZE - 1)
_CONST = _N * ((_SIZE - 1) * _FILL * math.log(_FILL) + _CONF * math.log(_CONF))

_NC, _NS = 2, 16
_NW = _NC * _NS

_R_SC = 1024
_R_TC = _N - _R_SC
_C_ALIGN = 99968
_RPW = _R_SC // _NW
_NSTRIPE = _RPW // 8

_CH_SIZES = [6400] * 15 + [3968]
_CH_OFFS = [sum(_CH_SIZES[:k]) for k in range(len(_CH_SIZES))]
_CH_MAX = max(_CH_SIZES)


def _tc_body(sliv_ref, tgt2_ref, o_ref):
    x2 = sliv_ref[...]
    t2 = tgt2_ref[...]
    cols2 = jax.lax.broadcasted_iota(jnp.int32, x2.shape, 1) + _C_ALIGN
    rows2 = jax.lax.broadcasted_iota(jnp.int32, x2.shape, 0)
    dense = jnp.sum(jnp.where(rows2 >= _R_TC, x2, jnp.float32(0.0)))
    corr = jnp.sum(jnp.where(cols2 == t2, x2, jnp.float32(0.0)))
    o_ref[0, 0] = (jnp.float32(_CONST)
                   - jnp.float32(_FILL) * dense
                   - jnp.float32(_CONF - _FILL) * corr)


def _tc_sum(sliver, tgt2d):
    return pl.pallas_call(
        _tc_body,
        out_specs=pl.BlockSpec(memory_space=pltpu.SMEM),
        out_shape=jax.ShapeDtypeStruct((1, 1), jnp.float32),
    )(sliver, tgt2d)


_sc_mesh = plsc.VectorSubcoreMesh(core_axis_name="c", subcore_axis_name="s")


@functools.partial(
    pl.kernel,
    mesh=_sc_mesh,
    out_type=jax.ShapeDtypeStruct((_NW, 16), jnp.float32),
    scratch_types=[
        pltpu.VMEM((48,), jnp.int32),
        pltpu.VMEM((2, 8, _CH_MAX), jnp.float32),
        pltpu.VMEM((16,), jnp.float32),
        pltpu.SemaphoreType.DMA,
        pltpu.SemaphoreType.DMA,
    ],
)
def _sc_part(x_hbm, tgt_hbm, out_hbm, tbuf, buf, stage, sem0, sem1):
    wid = lax.axis_index("s") * _NC + lax.axis_index("c")
    sems = (sem0, sem1)
    r0 = _R_TC + _RPW * wid

    pltpu.sync_copy(tgt_hbm.at[pl.ds(r0, _RPW)], tbuf.at[pl.ds(0, _RPW)])

    rows16 = lax.iota(jnp.int32, 16)
    zero16 = jnp.zeros((16,), jnp.float32)

    def _stripe(s, carry):
        g = carry[8]
        accs = carry[:8]
        rs = pl.multiple_of(r0 + 8 * s, 8)
        tv = tbuf[pl.ds(pl.multiple_of(8 * s, 8), 16)]
        ts = [tv[r] for r in range(8)]

        def _start(k):
            pltpu.async_copy(
                x_hbm.at[pl.ds(rs, 8), pl.ds(_CH_OFFS[k], _CH_SIZES[k])],
                buf.at[k % 2, :, pl.ds(0, _CH_SIZES[k])], sems[k % 2])

        _start(0)
        for k in range(len(_CH_SIZES)):
            if k + 1 < len(_CH_SIZES):
                _start(k + 1)
            slot = k % 2
            off, ch = _CH_OFFS[k], _CH_SIZES[k]
            pltpu.make_async_copy(
                x_hbm.at[pl.ds(rs, 8), pl.ds(off, ch)],
                buf.at[slot, :, pl.ds(0, ch)], sems[slot]).wait()

            def _ibody(jj, accs, slot=slot):
                jx = pl.multiple_of(jj * 32, 32)
                out = []
                for r in range(8):
                    out.append(accs[r] + buf[slot, r, pl.ds(jx, 16)]
                               + buf[slot, r, pl.ds(jx + 16, 16)])
                return tuple(out)

            accs = lax.fori_loop(0, ch // 32, _ibody, accs)

            for r in range(8):
                dt = ts[r] - jnp.int32(off)
                l0 = jnp.minimum(
                    jnp.maximum(dt & jnp.int32(-16), jnp.int32(0)),
                    jnp.int32(ch - 16))
                v = buf[slot, r, pl.ds(pl.multiple_of(l0, 16), 16)]
                lsel = jnp.where(dt >= 0,
                                 jnp.where(dt < ch, dt & jnp.int32(15),
                                           jnp.int32(16)),
                                 jnp.int32(16))
                g = g + jnp.where(rows16 == jnp.full((16,), lsel), v, zero16)
        return accs + (g,)

    init = tuple(jnp.zeros((16,), jnp.float32) for _ in range(9))
    res = lax.fori_loop(0, _NSTRIPE, _stripe, init)
    acc = ((res[0] + res[1]) + (res[2] + res[3])
           + ((res[4] + res[5]) + (res[6] + res[7])))
    stage[...] = (jnp.float32(_FILL) * acc
                  + jnp.float32(_CONF - _FILL) * res[8])
    pltpu.sync_copy(stage, out_hbm.at[wid])


def kernel(x, target):
    tgt = target.astype(jnp.int32)
    sliver = lax.slice(x, (0, _C_ALIGN), (_N, _SIZE))
    tc_out = _tc_sum(sliver, tgt.reshape(_N, 1))
    sc_out = _sc_part(x, tgt)
    return (tc_out[0, 0] - jnp.sum(sc_out)).reshape(())


# speedup vs baseline: 1.0488x; 1.0167x over previous
import functools
import math

import jax
import jax.numpy as jnp
from jax import lax
from jax.experimental import pallas as pl
from jax.experimental.pallas import tpu as pltpu
from jax.experimental.pallas import tpu_sc as plsc

_SIZE = 100000
_SMOOTHING = 0.1
_CONF = 1.0 - _SMOOTHING
_N = 1024
_FILL = _SMOOTHING / (_SIZE - 1)
_CONST = _N * ((_SIZE - 1) * _FILL * math.log(_FILL) + _CONF * math.log(_CONF))

_NC, _NS = 2, 16
_NW = _NC * _NS

_R_SC = 1024
_R_TC = _N - _R_SC
_C_ALIGN = 99968
_RPW = _R_SC // _NW
_NSTRIPE = _RPW // 8

_CH_SIZES = [4736] * 21 + [512]
_NBUF = 3
_CH_OFFS = [sum(_CH_SIZES[:k]) for k in range(len(_CH_SIZES))]
_CH_MAX = max(_CH_SIZES)


def _tc_body(sliv_ref, tgt2_ref, o_ref):
    x2 = sliv_ref[...]
    t2 = tgt2_ref[...]
    cols2 = jax.lax.broadcasted_iota(jnp.int32, x2.shape, 1) + _C_ALIGN
    rows2 = jax.lax.broadcasted_iota(jnp.int32, x2.shape, 0)
    dense = jnp.sum(jnp.where(rows2 >= _R_TC, x2, jnp.float32(0.0)))
    corr = jnp.sum(jnp.where(cols2 == t2, x2, jnp.float32(0.0)))
    o_ref[0, 0] = (jnp.float32(_CONST)
                   - jnp.float32(_FILL) * dense
                   - jnp.float32(_CONF - _FILL) * corr)


def _tc_sum(sliver, tgt2d):
    return pl.pallas_call(
        _tc_body,
        out_specs=pl.BlockSpec(memory_space=pltpu.SMEM),
        out_shape=jax.ShapeDtypeStruct((1, 1), jnp.float32),
    )(sliver, tgt2d)


_sc_mesh = plsc.VectorSubcoreMesh(core_axis_name="c", subcore_axis_name="s")


@functools.partial(
    pl.kernel,
    mesh=_sc_mesh,
    out_type=jax.ShapeDtypeStruct((_NW, 16), jnp.float32),
    scratch_types=[
        pltpu.VMEM((48,), jnp.int32),
        pltpu.VMEM((_NBUF, 8, _CH_MAX), jnp.float32),
        pltpu.VMEM((16,), jnp.float32),
        pltpu.SemaphoreType.DMA,
        pltpu.SemaphoreType.DMA,
        pltpu.SemaphoreType.DMA,
    ],
)
def _sc_part(x_hbm, tgt_hbm, out_hbm, tbuf, buf, stage, sem0, sem1, sem2):
    wid = lax.axis_index("s") * _NC + lax.axis_index("c")
    sems = (sem0, sem1, sem2)
    r0 = _R_TC + _RPW * wid

    pltpu.sync_copy(tgt_hbm.at[pl.ds(r0, _RPW)], tbuf.at[pl.ds(0, _RPW)])

    rows16 = lax.iota(jnp.int32, 16)
    zero16 = jnp.zeros((16,), jnp.float32)

    def _stripe(s, carry):
        g = carry[8]
        accs = carry[:8]
        rs = pl.multiple_of(r0 + 8 * s, 8)
        tv = tbuf[pl.ds(pl.multiple_of(8 * s, 8), 16)]
        ts = [tv[r] for r in range(8)]

        def _start(k):
            pltpu.async_copy(
                x_hbm.at[pl.ds(rs, 8), pl.ds(_CH_OFFS[k], _CH_SIZES[k])],
                buf.at[k % _NBUF, :, pl.ds(0, _CH_SIZES[k])], sems[k % _NBUF])

        for kp in range(_NBUF - 1):
            _start(kp)
        for k in range(len(_CH_SIZES)):
            if k + _NBUF - 1 < len(_CH_SIZES):
                _start(k + _NBUF - 1)
            slot = k % _NBUF
            off, ch = _CH_OFFS[k], _CH_SIZES[k]
            pltpu.make_async_copy(
                x_hbm.at[pl.ds(rs, 8), pl.ds(off, ch)],
                buf.at[slot, :, pl.ds(0, ch)], sems[slot]).wait()

            def _ibody(jj, accs, slot=slot):
                jx = pl.multiple_of(jj * 32, 32)
                out = []
                for r in range(8):
                    out.append(accs[r] + buf[slot, r, pl.ds(jx, 16)]
                               + buf[slot, r, pl.ds(jx + 16, 16)])
                return tuple(out)

            accs = lax.fori_loop(0, ch // 32, _ibody, accs)

            for r in range(8):
                dt = ts[r] - jnp.int32(off)
                l0 = jnp.minimum(
                    jnp.maximum(dt & jnp.int32(-16), jnp.int32(0)),
                    jnp.int32(ch - 16))
                v = buf[slot, r, pl.ds(pl.multiple_of(l0, 16), 16)]
                lsel = jnp.where(dt >= 0,
                                 jnp.where(dt < ch, dt & jnp.int32(15),
                                           jnp.int32(16)),
                                 jnp.int32(16))
                g = g + jnp.where(rows16 == jnp.full((16,), lsel), v, zero16)
        return accs + (g,)

    init = tuple(jnp.zeros((16,), jnp.float32) for _ in range(9))
    res = lax.fori_loop(0, _NSTRIPE, _stripe, init)
    acc = ((res[0] + res[1]) + (res[2] + res[3])
           + ((res[4] + res[5]) + (res[6] + res[7])))
    stage[...] = (jnp.float32(_FILL) * acc
                  + jnp.float32(_CONF - _FILL) * res[8])
    pltpu.sync_copy(stage, out_hbm.at[wid])


def kernel(x, target):
    tgt = target.astype(jnp.int32)
    sliver = lax.slice(x, (0, _C_ALIGN), (_N, _SIZE))
    tc_out = _tc_sum(sliver, tgt.reshape(_N, 1))
    sc_out = _sc_part(x, tgt)
    return (tc_out[0, 0] - jnp.sum(sc_out)).reshape(())


# speedup vs baseline: 1.0525x; 1.0035x over previous
import functools
import math

import jax
import jax.numpy as jnp
from jax import lax
from jax.experimental import pallas as pl
from jax.experimental.pallas import tpu as pltpu
from jax.experimental.pallas import tpu_sc as plsc

_SIZE = 100000
_SMOOTHING = 0.1
_CONF = 1.0 - _SMOOTHING
_N = 1024
_FILL = _SMOOTHING / (_SIZE - 1)
_CONST = _N * ((_SIZE - 1) * _FILL * math.log(_FILL) + _CONF * math.log(_CONF))

_NC, _NS = 2, 16
_NW = _NC * _NS

_R_SC = 1024
_R_TC = _N - _R_SC
_C_ALIGN = 99968
_RPW = _R_SC // _NW
_NSTRIPE = _RPW // 8

_CH_SIZES = [3712] * 26 + [3456]
_NBUF = 4
_CH_OFFS = [sum(_CH_SIZES[:k]) for k in range(len(_CH_SIZES))]
_CH_MAX = max(_CH_SIZES)


def _tc_body(sliv_ref, tgt2_ref, o_ref):
    x2 = sliv_ref[...]
    t2 = tgt2_ref[...]
    cols2 = jax.lax.broadcasted_iota(jnp.int32, x2.shape, 1) + _C_ALIGN
    rows2 = jax.lax.broadcasted_iota(jnp.int32, x2.shape, 0)
    dense = jnp.sum(jnp.where(rows2 >= _R_TC, x2, jnp.float32(0.0)))
    corr = jnp.sum(jnp.where(cols2 == t2, x2, jnp.float32(0.0)))
    o_ref[0, 0] = (jnp.float32(_CONST)
                   - jnp.float32(_FILL) * dense
                   - jnp.float32(_CONF - _FILL) * corr)


def _tc_sum(sliver, tgt2d):
    return pl.pallas_call(
        _tc_body,
        out_specs=pl.BlockSpec(memory_space=pltpu.SMEM),
        out_shape=jax.ShapeDtypeStruct((1, 1), jnp.float32),
    )(sliver, tgt2d)


_sc_mesh = plsc.VectorSubcoreMesh(core_axis_name="c", subcore_axis_name="s")


@functools.partial(
    pl.kernel,
    mesh=_sc_mesh,
    out_type=jax.ShapeDtypeStruct((_NW, 16), jnp.float32),
    scratch_types=[
        pltpu.VMEM((48,), jnp.int32),
        pltpu.VMEM((_NBUF, 8, _CH_MAX), jnp.float32),
        pltpu.VMEM((16,), jnp.float32),
        pltpu.SemaphoreType.DMA,
        pltpu.SemaphoreType.DMA,
        pltpu.SemaphoreType.DMA,
        pltpu.SemaphoreType.DMA,
    ],
)
def _sc_part(x_hbm, tgt_hbm, out_hbm, tbuf, buf, stage, sem0, sem1, sem2, sem3):
    wid = lax.axis_index("s") * _NC + lax.axis_index("c")
    sems = (sem0, sem1, sem2, sem3)
    r0 = _R_TC + _RPW * wid

    pltpu.sync_copy(tgt_hbm.at[pl.ds(r0, _RPW)], tbuf.at[pl.ds(0, _RPW)])

    rows16 = lax.iota(jnp.int32, 16)
    zero16 = jnp.zeros((16,), jnp.float32)

    def _stripe(s, carry):
        g = carry[8]
        accs = carry[:8]
        rs = pl.multiple_of(r0 + 8 * s, 8)
        tv = tbuf[pl.ds(pl.multiple_of(8 * s, 8), 16)]
        ts = [tv[r] for r in range(8)]

        def _start(k):
            pltpu.async_copy(
                x_hbm.at[pl.ds(rs, 8), pl.ds(_CH_OFFS[k], _CH_SIZES[k])],
                buf.at[k % _NBUF, :, pl.ds(0, _CH_SIZES[k])], sems[k % _NBUF])

        for kp in range(_NBUF - 1):
            _start(kp)
        for k in range(len(_CH_SIZES)):
            if k + _NBUF - 1 < len(_CH_SIZES):
                _start(k + _NBUF - 1)
            slot = k % _NBUF
            off, ch = _CH_OFFS[k], _CH_SIZES[k]
            pltpu.make_async_copy(
                x_hbm.at[pl.ds(rs, 8), pl.ds(off, ch)],
                buf.at[slot, :, pl.ds(0, ch)], sems[slot]).wait()

            def _ibody(jj, accs, slot=slot):
                jx = pl.multiple_of(jj * 32, 32)
                out = []
                for r in range(8):
                    out.append(accs[r] + buf[slot, r, pl.ds(jx, 16)]
                               + buf[slot, r, pl.ds(jx + 16, 16)])
                return tuple(out)

            accs = lax.fori_loop(0, ch // 32, _ibody, accs)

            for r in range(8):
                dt = ts[r] - jnp.int32(off)
                l0 = jnp.minimum(
                    jnp.maximum(dt & jnp.int32(-16), jnp.int32(0)),
                    jnp.int32(ch - 16))
                v = buf[slot, r, pl.ds(pl.multiple_of(l0, 16), 16)]
                lsel = jnp.where(dt >= 0,
                                 jnp.where(dt < ch, dt & jnp.int32(15),
                                           jnp.int32(16)),
                                 jnp.int32(16))
                g = g + jnp.where(rows16 == jnp.full((16,), lsel), v, zero16)
        return accs + (g,)

    init = tuple(jnp.zeros((16,), jnp.float32) for _ in range(9))
    res = lax.fori_loop(0, _NSTRIPE, _stripe, init)
    acc = ((res[0] + res[1]) + (res[2] + res[3])
           + ((res[4] + res[5]) + (res[6] + res[7])))
    stage[...] = (jnp.float32(_FILL) * acc
                  + jnp.float32(_CONF - _FILL) * res[8])
    pltpu.sync_copy(stage, out_hbm.at[wid])


def kernel(x, target):
    tgt = target.astype(jnp.int32)
    sliver = lax.slice(x, (0, _C_ALIGN), (_N, _SIZE))
    tc_out = _tc_sum(sliver, tgt.reshape(_N, 1))
    sc_out = _sc_part(x, tgt)
    return (tc_out[0, 0] - jnp.sum(sc_out)).reshape(())
